# baseline (speedup 1.0000x reference)
import functools

import jax
import jax.numpy as jnp
from jax import lax
from jax.experimental import pallas as pl
from jax.experimental.pallas import tpu as pltpu
from jax.experimental.pallas import tpu_sc as plsc

_VOCAB = 1000000
_DIM = 64
_BATCH = 4096
_HIST = 200

_B = _BATCH * _HIST
_NC = 2
_NS = 16
_NW = _NC * _NS
_B_PER_W = _B // _NW
_CHUNK = 1024
_N_CHUNKS = _B_PER_W // _CHUNK

_mesh = plsc.VectorSubcoreMesh(core_axis_name="c", subcore_axis_name="s")


@functools.partial(
    pl.kernel,
    mesh=_mesh,
    out_type=jax.ShapeDtypeStruct((_B, _DIM), jnp.float32),
    compiler_params=pltpu.CompilerParams(use_tc_tiling_on_sc=False),
    scratch_types=[
        pltpu.VMEM((_CHUNK,), jnp.int32),
        pltpu.VMEM((_CHUNK, _DIM), jnp.float32),
        pltpu.SemaphoreType.DMA,
    ],
)
def _emb_lookup(idx_hbm, table_hbm, out_hbm, idx_v, rows_v, sem):
    wid = lax.axis_index("s") * _NC + lax.axis_index("c")
    base = wid * _B_PER_W

    def body(i, carry):
        off = base + i * _CHUNK
        pltpu.sync_copy(idx_hbm.at[pl.ds(off, _CHUNK)], idx_v)
        pltpu.async_copy(table_hbm.at[idx_v], rows_v, sem).wait()
        pltpu.sync_copy(rows_v, out_hbm.at[pl.ds(off, _CHUNK)])
        return carry

    lax.fori_loop(0, _N_CHUNKS, body, 0)


def kernel(x, emb_weight):
    flat = x.reshape(_B)
    out = _emb_lookup(flat, emb_weight)
    return out.reshape(_BATCH, _HIST, _DIM)


# speedup vs baseline: 1.0101x; 1.0101x over previous
import functools

import jax
import jax.numpy as jnp
from jax import lax
from jax.experimental import pallas as pl
from jax.experimental.pallas import tpu as pltpu
from jax.experimental.pallas import tpu_sc as plsc

_VOCAB = 1000000
_DIM = 64
_BATCH = 4096
_HIST = 200

_B = _BATCH * _HIST
_NC = 2
_NS = 16
_NW = _NC * _NS
_B_PER_W = _B // _NW
_CHUNK = 512
_N_CHUNKS = _B_PER_W // _CHUNK
_NBUF = 2

_mesh = plsc.VectorSubcoreMesh(core_axis_name="c", subcore_axis_name="s")


@functools.partial(
    pl.kernel,
    mesh=_mesh,
    out_type=jax.ShapeDtypeStruct((_B, _DIM), jnp.float32),
    compiler_params=pltpu.CompilerParams(use_tc_tiling_on_sc=False),
    scratch_types=[
        pltpu.VMEM((_B_PER_W,), jnp.int32),
        pltpu.VMEM((_NBUF, _CHUNK, _DIM), jnp.float32),
        pltpu.SemaphoreType.DMA,
        pltpu.SemaphoreType.DMA,
        pltpu.SemaphoreType.DMA,
        pltpu.SemaphoreType.DMA,
    ],
)
def _emb_lookup(idx_hbm, table_hbm, out_hbm, idx_v, rows_v, g0, g1, o0, o1):
    g_sems = [g0, g1]
    o_sems = [o0, o1]
    wid = lax.axis_index("s") * _NC + lax.axis_index("c")
    base = wid * _B_PER_W

    pltpu.sync_copy(idx_hbm.at[pl.ds(base, _B_PER_W)], idx_v)

    def gather_desc(i, b):
        return pltpu.make_async_copy(
            table_hbm.at[idx_v.at[pl.ds(i * _CHUNK, _CHUNK)]],
            rows_v.at[b],
            g_sems[b],
        )

    def out_desc(i, b):
        return pltpu.make_async_copy(
            rows_v.at[b],
            out_hbm.at[pl.ds(base + i * _CHUNK, _CHUNK)],
            o_sems[b],
        )

    for b in range(_NBUF):
        gather_desc(b, b).start()

    @pl.loop(0, _N_CHUNKS, step=_NBUF)
    def body(g):
        for b in range(_NBUF):
            i = g + b
            gather_desc(i, b).wait()
            out_desc(i, b).start()
        for b in range(_NBUF):
            i = g + b + _NBUF

            @pl.when(i < _N_CHUNKS)
            def _():
                out_desc(i - _NBUF, b).wait()
                gather_desc(i, b).start()

    for b in range(_NBUF):
        out_desc(_N_CHUNKS - _NBUF + b, b).wait()


def kernel(x, emb_weight):
    flat = x.reshape(_B)
    out = _emb_lookup(flat, emb_weight)
    return out.reshape(_BATCH, _HIST, _DIM)


# speedup vs baseline: 1.0126x; 1.0026x over previous
import functools

import jax
import jax.numpy as jnp
from jax import lax
from jax.experimental import pallas as pl
from jax.experimental.pallas import tpu as pltpu
from jax.experimental.pallas import tpu_sc as plsc

_VOCAB = 1000000
_DIM = 64
_BATCH = 4096
_HIST = 200

_NC = 2
_NS = 16
_NW = _NC * _NS
_ROWS_PER_W = _BATCH // _NW
_NBUF = 4

_mesh = plsc.VectorSubcoreMesh(core_axis_name="c", subcore_axis_name="s")


@functools.partial(
    pl.kernel,
    mesh=_mesh,
    out_type=jax.ShapeDtypeStruct((_BATCH, _HIST, _DIM), jnp.float32),
    compiler_params=pltpu.CompilerParams(use_tc_tiling_on_sc=False),
    scratch_types=[
        pltpu.VMEM((_ROWS_PER_W, _HIST), jnp.int32),
        pltpu.VMEM((_NBUF, _HIST, _DIM), jnp.float32),
        pltpu.SemaphoreType.DMA,
        pltpu.SemaphoreType.DMA,
        pltpu.SemaphoreType.DMA,
        pltpu.SemaphoreType.DMA,
        pltpu.SemaphoreType.DMA,
        pltpu.SemaphoreType.DMA,
        pltpu.SemaphoreType.DMA,
        pltpu.SemaphoreType.DMA,
    ],
)
def _emb_lookup(x_hbm, table_hbm, out_hbm, idx_v, rows_v, *sems):
    g_sems = sems[:_NBUF]
    o_sems = sems[_NBUF:]
    wid = lax.axis_index("s") * _NC + lax.axis_index("c")
    row0 = wid * _ROWS_PER_W

    pltpu.sync_copy(x_hbm.at[pl.ds(row0, _ROWS_PER_W), :], idx_v)

    def gather_desc(r, b):
        return pltpu.make_async_copy(
            table_hbm.at[idx_v.at[r]],
            rows_v.at[b],
            g_sems[b],
        )

    def out_desc(r, b):
        return pltpu.make_async_copy(
            rows_v.at[b],
            out_hbm.at[row0 + r],
            o_sems[b],
        )

    for b in range(_NBUF):
        gather_desc(b, b).start()

    @pl.loop(0, _ROWS_PER_W, step=_NBUF)
    def body(g):
        for b in range(_NBUF):
            r = g + b
            gather_desc(r, b).wait()
            out_desc(r, b).start()
        for b in range(_NBUF):
            r = g + b + _NBUF

            @pl.when(r < _ROWS_PER_W)
            def _():
                out_desc(r - _NBUF, b).wait()
                gather_desc(r, b).start()

    for b in range(_NBUF):
        out_desc(_ROWS_PER_W - _NBUF + b, b).wait()


def kernel(x, emb_weight):
    return _emb_lookup(x, emb_weight)


# speedup vs baseline: 1.3446x; 1.3278x over previous
import functools

import jax
import jax.numpy as jnp
from jax import lax
from jax.experimental import pallas as pl
from jax.experimental.pallas import tpu as pltpu
from jax.experimental.pallas import tpu_sc as plsc

_VOCAB = 1000000
_DIM = 64
_BATCH = 4096
_HIST = 200

_B = _BATCH * _HIST
_NC = 2
_NS = 16
_NW = _NC * _NS
_B_PER_W = _B // _NW
_CHUNK = 512
_N_CHUNKS = _B_PER_W // _CHUNK
_NBUF = 2

_mesh = plsc.VectorSubcoreMesh(core_axis_name="c", subcore_axis_name="s")


@functools.partial(
    pl.kernel,
    mesh=_mesh,
    out_type=jax.ShapeDtypeStruct((_B, 2 * _DIM), jnp.float32),
    compiler_params=pltpu.CompilerParams(use_tc_tiling_on_sc=False),
    scratch_types=[
        pltpu.VMEM((_B_PER_W,), jnp.int32),
        pltpu.VMEM((_NBUF, _CHUNK, _DIM), jnp.float32),
        pltpu.SemaphoreType.DMA,
        pltpu.SemaphoreType.DMA,
        pltpu.SemaphoreType.DMA,
        pltpu.SemaphoreType.DMA,
    ],
)
def _emb_lookup(idx_hbm, table_hbm, out_hbm, idx_v, rows_v, g0, g1, o0, o1):
    g_sems = [g0, g1]
    o_sems = [o0, o1]
    wid = lax.axis_index("s") * _NC + lax.axis_index("c")
    base = wid * _B_PER_W

    pltpu.sync_copy(idx_hbm.at[pl.ds(base, _B_PER_W)], idx_v)

    def gather_desc(i, b):
        return pltpu.make_async_copy(
            table_hbm.at[idx_v.at[pl.ds(i * _CHUNK, _CHUNK)]],
            rows_v.at[b],
            g_sems[b],
        )

    def out_desc(i, b):
        return pltpu.make_async_copy(
            rows_v.at[b],
            out_hbm.at[pl.ds(base + i * _CHUNK, _CHUNK), pl.ds(0, _DIM)],
            o_sems[b],
        )

    for b in range(_NBUF):
        gather_desc(b, b).start()

    @pl.loop(0, _N_CHUNKS, step=_NBUF)
    def body(g):
        for b in range(_NBUF):
            i = g + b
            gather_desc(i, b).wait()
            out_desc(i, b).start()
        for b in range(_NBUF):
            i = g + b + _NBUF

            @pl.when(i < _N_CHUNKS)
            def _():
                out_desc(i - _NBUF, b).wait()
                gather_desc(i, b).start()

    for b in range(_NBUF):
        out_desc(_N_CHUNKS - _NBUF + b, b).wait()


def kernel(x, emb_weight):
    flat = x.reshape(_B)
    out2 = _emb_lookup(flat, emb_weight)
    return out2[:, :_DIM].reshape(_BATCH, _HIST, _DIM)
